# baseline (speedup 1.0000x reference)
import dataclasses
import functools

import jax
import jax.numpy as jnp
from jax import lax
from jax.experimental import pallas as pl
from jax.experimental.pallas import tpu as pltpu
from jax.experimental.pallas import tpu_sc as plsc

T = 2048
D = 1024
F = 2048
E = 8

BLK = 256
NB = T // BLK + E
PAD_T = NB * BLK
NC, NS, L = 2, 16, 16
NW = NC * NS
CH = PAD_T // NW
QR = CH // 4
NCHUNK = T // L

_vector_mesh = plsc.VectorSubcoreMesh(core_axis_name="c", subcore_axis_name="s")

_sc_params = pltpu.CompilerParams()
if "needs_layout_passes" in pltpu.CompilerParams.__dataclass_fields__:
    _sc_params = dataclasses.replace(_sc_params, needs_layout_passes=False)


def _gating_body(x_ref, gw_ref, gb_ref, top_ref, wt_ref):
    logits = jnp.dot(x_ref[...], gw_ref[...], preferred_element_type=jnp.float32)
    logits = logits + gb_ref[...]
    m = jnp.max(logits, axis=-1, keepdims=True)
    ssum = jnp.sum(jnp.exp(logits - m), axis=-1)
    top = jnp.argmax(logits, axis=-1).astype(jnp.int32)
    top_ref[...] = top[None, :]
    wt_ref[...] = (1.0 / ssum)[None, :]


def _route_gather_body(top_hbm, wt_hbm, x_hbm,
                       xs_hbm, dst_hbm, ws_hbm, be_hbm,
                       top_v, w_v, src_v, dst_v, ws_v, be_v, s_v, rows_v,
                       sem, sem2):
    wid = lax.axis_index("s") * NC + lax.axis_index("c")

    pltpu.sync_copy(top_hbm, top_v)
    pltpu.sync_copy(wt_hbm, w_v)

    zeros = jnp.zeros((L,), jnp.int32)
    trash = jnp.full((L,), T, jnp.int32)
    fzeros = jnp.zeros((L,), jnp.float32)

    @pl.loop(0, PAD_T, step=L)
    def _(i):
        src_v[pl.ds(i, L)] = zeros
        dst_v[pl.ds(i, L)] = trash
        ws_v[pl.ds(i, L)] = fzeros

    def count_body(i, accs):
        v = top_v[pl.ds(i * L, L)]
        return tuple(
            accs[e] + (v == e).astype(jnp.int32) for e in range(E)
        )

    accs = lax.fori_loop(0, NCHUNK, count_body,
                         tuple(jnp.zeros((L,), jnp.int32) for _ in range(E)))
    counts = [jnp.sum(accs[e]) for e in range(E)]

    bases = []
    ends = []
    cum_blocks = jnp.int32(0)
    for e in range(E):
        bases.append(cum_blocks * BLK)
        cum_blocks = cum_blocks + (counts[e] + (BLK - 1)) // BLK
        ends.append(cum_blocks)

    biota = lax.iota(jnp.int32, L)
    for chunk in range(2):
        bvec = biota + chunk * L
        acc = jnp.zeros((L,), jnp.int32)
        for e in range(E):
            acc = acc + (bvec >= ends[e]).astype(jnp.int32)
        bev = jnp.minimum(acc, E - 1)
        if chunk == 1:
            bev = jnp.where(biota == L - 1, cum_blocks, bev)
        be_v[pl.ds(chunk * L, L)] = bev

    base_vec = jnp.zeros((L,), jnp.int32)
    for e in range(E):
        base_vec = jnp.where(biota == e, bases[e], base_vec)
    s_v[...] = base_vec

    @pl.loop(0, NCHUNK)
    def _(i):
        v = top_v[pl.ds(i * L, L)]
        wv = w_v[pl.ds(i * L, L)]
        tok = lax.iota(jnp.int32, L) + i * L
        g = plsc.load_gather(s_v, [v])
        r, last = plsc.scan_count(v)
        pos = g + r - 1
        plsc.store_scatter(src_v, [pos], tok)
        plsc.store_scatter(dst_v, [pos], tok)
        plsc.store_scatter(ws_v, [pos], wv)
        plsc.store_scatter(s_v, [v], pos + 1, mask=last)

    base = wid * CH
    gathers = []
    writes = []
    for k in range(4):
        start = base + k * QR
        buf = rows_v.at[k % 2]
        if k >= 2:
            writes[k - 2].wait()
        gathers.append(
            pltpu.async_copy(x_hbm.at[src_v.at[pl.ds(start, QR)]], buf,
                             sem if k % 2 == 0 else sem2))
        gathers[k].wait()
        writes.append(
            pltpu.async_copy(buf, xs_hbm.at[pl.ds(start, QR)],
                             sem if k % 2 == 0 else sem2))
    pltpu.sync_copy(dst_v.at[pl.ds(base, CH)], dst_hbm.at[pl.ds(base, CH)])
    pltpu.sync_copy(ws_v.at[pl.ds(base, CH)], ws_hbm.at[pl.ds(base, CH)])
    writes[2].wait()
    writes[3].wait()

    @pl.when(wid == 0)
    def _():
        pltpu.sync_copy(be_v, be_hbm)


def _gelu_exact(v):
    return 0.5 * v * (1.0 + lax.erf(v * 0.7071067811865476))


def _ffn_body(be_ref, x_ref, w1_ref, w2_ref, w3_ref, b1_ref, b2_ref, b3_ref,
              ws_ref, out_ref):
    @pl.when(pl.program_id(0) < be_ref[2 * L - 1])
    def _():
        xb = x_ref[...]
        h1 = jnp.dot(xb, w1_ref[0],
                     preferred_element_type=jnp.float32) + b1_ref[0]
        h2 = jnp.dot(xb, w2_ref[0],
                     preferred_element_type=jnp.float32) + b2_ref[0]
        h = _gelu_exact(h1) * h2
        o = jnp.dot(h, w3_ref[0], preferred_element_type=jnp.float32) + b3_ref[0]
        out_ref[...] = o * ws_ref[...]


def _scatter_body(osort_hbm, dst_hbm, out_hbm, idx_v, rows_v, sem, sem2):
    wid = lax.axis_index("s") * NC + lax.axis_index("c")
    base = wid * CH
    reads = []
    scats = []
    for k in range(4):
        start = base + k * QR
        buf = rows_v.at[k % 2]
        s = sem if k % 2 == 0 else sem2
        pltpu.sync_copy(dst_hbm.at[pl.ds(start, QR)], idx_v.at[k])
        if k >= 2:
            scats[k - 2].wait()
        reads.append(pltpu.async_copy(osort_hbm.at[pl.ds(start, QR)], buf, s))
        reads[k].wait()
        scats.append(pltpu.async_copy(buf, out_hbm.at[idx_v.at[k]], s))
    scats[2].wait()
    scats[3].wait()


@jax.jit
def _moe(x, gate_w, gate_b, w1, b1, w2, b2, w3, b3):
    xt = x.reshape(T, D)
    gb = gate_b.reshape(1, E)

    top, wt = pl.pallas_call(
        _gating_body,
        grid=(8,),
        out_shape=(
            jax.ShapeDtypeStruct((1, T), jnp.int32),
            jax.ShapeDtypeStruct((1, T), jnp.float32),
        ),
        in_specs=[
            pl.BlockSpec((T // 8, D), lambda t: (t, 0)),
            pl.BlockSpec((D, E), lambda t: (0, 0)),
            pl.BlockSpec((1, E), lambda t: (0, 0)),
        ],
        out_specs=(
            pl.BlockSpec((1, T // 8), lambda t: (0, t)),
            pl.BlockSpec((1, T // 8), lambda t: (0, t)),
        ),
        compiler_params=pltpu.CompilerParams(
            dimension_semantics=("arbitrary",),
        ),
    )(xt, gate_w, gb)

    route = pl.kernel(
        _route_gather_body,
        out_type=(
            jax.ShapeDtypeStruct((PAD_T, D), jnp.float32),
            jax.ShapeDtypeStruct((PAD_T,), jnp.int32),
            jax.ShapeDtypeStruct((PAD_T,), jnp.float32),
            jax.ShapeDtypeStruct((2 * L,), jnp.int32),
        ),
        mesh=_vector_mesh,
        scratch_types=[
            pltpu.VMEM((T,), jnp.int32),
            pltpu.VMEM((T,), jnp.float32),
            pltpu.VMEM((PAD_T,), jnp.int32),
            pltpu.VMEM((PAD_T,), jnp.int32),
            pltpu.VMEM((PAD_T,), jnp.float32),
            pltpu.VMEM((2 * L,), jnp.int32),
            pltpu.VMEM((L,), jnp.int32),
            pltpu.VMEM((2, QR, D), jnp.float32),
            pltpu.SemaphoreType.DMA,
            pltpu.SemaphoreType.DMA,
        ],
        compiler_params=_sc_params,
    )
    x_sorted, dst_ids, ws, bexp = route(top.reshape(T), wt.reshape(T), xt)

    b1r = b1.reshape(E, 1, F)
    b2r = b2.reshape(E, 1, F)
    b3r = b3.reshape(E, 1, D)
    ws2 = ws.reshape(PAD_T, 1)

    grid_spec = pltpu.PrefetchScalarGridSpec(
        num_scalar_prefetch=1,
        grid=(NB,),
        in_specs=[
            pl.BlockSpec((BLK, D), lambda b, be: (b, 0)),
            pl.BlockSpec((1, D, F), lambda b, be: (be[b], 0, 0)),
            pl.BlockSpec((1, D, F), lambda b, be: (be[b], 0, 0)),
            pl.BlockSpec((1, F, D), lambda b, be: (be[b], 0, 0)),
            pl.BlockSpec((1, 1, F), lambda b, be: (be[b], 0, 0)),
            pl.BlockSpec((1, 1, F), lambda b, be: (be[b], 0, 0)),
            pl.BlockSpec((1, 1, D), lambda b, be: (be[b], 0, 0)),
            pl.BlockSpec((BLK, 1), lambda b, be: (b, 0)),
        ],
        out_specs=pl.BlockSpec((BLK, D), lambda b, be: (b, 0)),
    )
    out_sorted = pl.pallas_call(
        _ffn_body,
        grid_spec=grid_spec,
        out_shape=jax.ShapeDtypeStruct((PAD_T, D), jnp.float32),
        compiler_params=pltpu.CompilerParams(
            dimension_semantics=("arbitrary",),
        ),
    )(bexp, x_sorted, w1, w2, w3, b1r, b2r, b3r, ws2)

    scatter = pl.kernel(
        _scatter_body,
        out_type=jax.ShapeDtypeStruct((T + 8, D), jnp.float32),
        mesh=_vector_mesh,
        scratch_types=[
            pltpu.VMEM((4, QR), jnp.int32),
            pltpu.VMEM((2, QR, D), jnp.float32),
            pltpu.SemaphoreType.DMA,
            pltpu.SemaphoreType.DMA,
        ],
        compiler_params=_sc_params,
    )
    out_padded = scatter(out_sorted, dst_ids)

    final = out_padded[:T].reshape(1, T, D)
    aux_loss = jnp.asarray(0.0, dtype=jnp.float32)
    return (final, aux_loss)


def kernel(x, gate_w, gate_b, w1, b1, w2, b2, w3, b3):
    return _moe(x, gate_w, gate_b, w1, b1, w2, b2, w3, b3)


# speedup vs baseline: 1.3446x; 1.3446x over previous
import dataclasses
import functools

import jax
import jax.numpy as jnp
from jax import lax
from jax.experimental import pallas as pl
from jax.experimental.pallas import tpu as pltpu
from jax.experimental.pallas import tpu_sc as plsc

T = 2048
D = 1024
F = 2048
E = 8

BLK = 128
NB = T // BLK + E
PAD_T = NB * BLK
NC, NS, L = 2, 16, 16
NW = NC * NS
CH = PAD_T // NW
HR = CH // 2
NCHUNK = T // L

_vector_mesh = plsc.VectorSubcoreMesh(core_axis_name="c", subcore_axis_name="s")

_sc_params = pltpu.CompilerParams()
if "needs_layout_passes" in pltpu.CompilerParams.__dataclass_fields__:
    _sc_params = dataclasses.replace(_sc_params, needs_layout_passes=False)


def _gating_body(x_ref, gw_ref, gb_ref, top_ref, wt_ref):
    logits = jnp.dot(x_ref[...], gw_ref[...], preferred_element_type=jnp.float32)
    logits = logits + gb_ref[...]
    m = jnp.max(logits, axis=-1, keepdims=True)
    ssum = jnp.sum(jnp.exp(logits - m), axis=-1)
    top = jnp.argmax(logits, axis=-1).astype(jnp.int32)
    top_ref[...] = top[None, :]
    wt_ref[...] = (1.0 / ssum)[None, :]


def _route_gather_body(top_hbm, wt_hbm, x_hbm,
                       xs_hbm, dst_hbm, ws_hbm, be_hbm,
                       top_v, w_v, src_v, dst_v, ws_v, be_v, s_v, rows_v,
                       sem, sem2):
    wid = lax.axis_index("s") * NC + lax.axis_index("c")

    pltpu.sync_copy(top_hbm, top_v)
    pltpu.sync_copy(wt_hbm, w_v)

    zeros = jnp.zeros((L,), jnp.int32)
    trash = jnp.full((L,), T, jnp.int32)
    fzeros = jnp.zeros((L,), jnp.float32)

    @pl.loop(0, PAD_T, step=L)
    def _(i):
        src_v[pl.ds(i, L)] = zeros
        dst_v[pl.ds(i, L)] = trash
        ws_v[pl.ds(i, L)] = fzeros

    def count_body(i, accs):
        v = top_v[pl.ds(i * L, L)]
        return tuple(
            accs[e] + (v == e).astype(jnp.int32) for e in range(E)
        )

    accs = lax.fori_loop(0, NCHUNK, count_body,
                         tuple(jnp.zeros((L,), jnp.int32) for _ in range(E)))
    counts = [jnp.sum(accs[e]) for e in range(E)]

    bases = []
    ends = []
    cum_blocks = jnp.int32(0)
    for e in range(E):
        bases.append(cum_blocks * BLK)
        cum_blocks = cum_blocks + (counts[e] + (BLK - 1)) // BLK
        ends.append(cum_blocks)

    biota = lax.iota(jnp.int32, L)
    for chunk in range(2):
        bvec = biota + chunk * L
        acc = jnp.zeros((L,), jnp.int32)
        for e in range(E):
            acc = acc + (bvec >= ends[e]).astype(jnp.int32)
        bev = jnp.minimum(acc, E - 1)
        if chunk == 1:
            bev = jnp.where(biota == L - 1, cum_blocks, bev)
        be_v[pl.ds(chunk * L, L)] = bev

    base_vec = jnp.zeros((L,), jnp.int32)
    for e in range(E):
        base_vec = jnp.where(biota == e, bases[e], base_vec)
    s_v[...] = base_vec

    @pl.loop(0, NCHUNK)
    def _(i):
        v = top_v[pl.ds(i * L, L)]
        wv = w_v[pl.ds(i * L, L)]
        tok = lax.iota(jnp.int32, L) + i * L
        g = plsc.load_gather(s_v, [v])
        r, last = plsc.scan_count(v)
        pos = g + r - 1
        plsc.store_scatter(src_v, [pos], tok)
        plsc.store_scatter(dst_v, [pos], tok)
        plsc.store_scatter(ws_v, [pos], wv)
        plsc.store_scatter(s_v, [v], pos + 1, mask=last)

    base = wid * CH
    g0 = pltpu.async_copy(x_hbm.at[src_v.at[pl.ds(base, HR)]],
                          rows_v.at[0], sem)
    g1 = pltpu.async_copy(x_hbm.at[src_v.at[pl.ds(base + HR, HR)]],
                          rows_v.at[1], sem2)
    g0.wait()
    w0 = pltpu.async_copy(rows_v.at[0], xs_hbm.at[pl.ds(base, HR)], sem)
    g1.wait()
    w1 = pltpu.async_copy(rows_v.at[1], xs_hbm.at[pl.ds(base + HR, HR)], sem2)
    pltpu.sync_copy(dst_v.at[pl.ds(base, CH)], dst_hbm.at[pl.ds(base, CH)])
    pltpu.sync_copy(ws_v.at[pl.ds(base, CH)], ws_hbm.at[pl.ds(base, CH)])
    w0.wait()
    w1.wait()

    @pl.when(wid == 0)
    def _():
        pltpu.sync_copy(be_v, be_hbm)


def _gelu_exact(v):
    return 0.5 * v * (1.0 + lax.erf(v * 0.7071067811865476))


def _ffn_body(be_ref, x_ref, w1_ref, w2_ref, w3_ref, b1_ref, b2_ref, b3_ref,
              ws_ref, out_ref):
    @pl.when(pl.program_id(0) < be_ref[2 * L - 1])
    def _():
        xb = x_ref[...]
        h1 = jnp.dot(xb, w1_ref[0],
                     preferred_element_type=jnp.float32) + b1_ref[0]
        h2 = jnp.dot(xb, w2_ref[0],
                     preferred_element_type=jnp.float32) + b2_ref[0]
        h = _gelu_exact(h1) * h2
        o = jnp.dot(h, w3_ref[0], preferred_element_type=jnp.float32) + b3_ref[0]
        out_ref[...] = o * ws_ref[...]


def _scatter_body(osort_hbm, dst_hbm, out_hbm, idx_v, rows_v, sem, sem2):
    wid = lax.axis_index("s") * NC + lax.axis_index("c")
    base = wid * CH
    r0 = pltpu.async_copy(osort_hbm.at[pl.ds(base, HR)], rows_v.at[0], sem)
    r1 = pltpu.async_copy(osort_hbm.at[pl.ds(base + HR, HR)], rows_v.at[1],
                          sem2)
    pltpu.sync_copy(dst_hbm.at[pl.ds(base, HR)], idx_v.at[0])
    pltpu.sync_copy(dst_hbm.at[pl.ds(base + HR, HR)], idx_v.at[1])
    r0.wait()
    s0 = pltpu.async_copy(rows_v.at[0], out_hbm.at[idx_v.at[0]], sem)
    r1.wait()
    s1 = pltpu.async_copy(rows_v.at[1], out_hbm.at[idx_v.at[1]], sem2)
    s0.wait()
    s1.wait()


@jax.jit
def _moe(x, gate_w, gate_b, w1, b1, w2, b2, w3, b3):
    xt = x.reshape(T, D)
    gb = gate_b.reshape(1, E)

    top, wt = pl.pallas_call(
        _gating_body,
        grid=(8,),
        out_shape=(
            jax.ShapeDtypeStruct((1, T), jnp.int32),
            jax.ShapeDtypeStruct((1, T), jnp.float32),
        ),
        in_specs=[
            pl.BlockSpec((T // 8, D), lambda t: (t, 0)),
            pl.BlockSpec((D, E), lambda t: (0, 0)),
            pl.BlockSpec((1, E), lambda t: (0, 0)),
        ],
        out_specs=(
            pl.BlockSpec((1, T // 8), lambda t: (0, t)),
            pl.BlockSpec((1, T // 8), lambda t: (0, t)),
        ),
        compiler_params=pltpu.CompilerParams(
            dimension_semantics=("parallel",),
        ),
    )(xt, gate_w, gb)

    route = pl.kernel(
        _route_gather_body,
        out_type=(
            jax.ShapeDtypeStruct((PAD_T, D), jnp.float32),
            jax.ShapeDtypeStruct((PAD_T,), jnp.int32),
            jax.ShapeDtypeStruct((PAD_T,), jnp.float32),
            jax.ShapeDtypeStruct((2 * L,), jnp.int32),
        ),
        mesh=_vector_mesh,
        scratch_types=[
            pltpu.VMEM((T,), jnp.int32),
            pltpu.VMEM((T,), jnp.float32),
            pltpu.VMEM((PAD_T,), jnp.int32),
            pltpu.VMEM((PAD_T,), jnp.int32),
            pltpu.VMEM((PAD_T,), jnp.float32),
            pltpu.VMEM((2 * L,), jnp.int32),
            pltpu.VMEM((L,), jnp.int32),
            pltpu.VMEM((2, HR, D), jnp.float32),
            pltpu.SemaphoreType.DMA,
            pltpu.SemaphoreType.DMA,
        ],
        compiler_params=_sc_params,
    )
    x_sorted, dst_ids, ws, bexp = route(top.reshape(T), wt.reshape(T), xt)

    b1r = b1.reshape(E, 1, F)
    b2r = b2.reshape(E, 1, F)
    b3r = b3.reshape(E, 1, D)
    ws2 = ws.reshape(PAD_T, 1)

    grid_spec = pltpu.PrefetchScalarGridSpec(
        num_scalar_prefetch=1,
        grid=(NB,),
        in_specs=[
            pl.BlockSpec((BLK, D), lambda b, be: (b, 0)),
            pl.BlockSpec((1, D, F), lambda b, be: (be[b], 0, 0)),
            pl.BlockSpec((1, D, F), lambda b, be: (be[b], 0, 0)),
            pl.BlockSpec((1, F, D), lambda b, be: (be[b], 0, 0)),
            pl.BlockSpec((1, 1, F), lambda b, be: (be[b], 0, 0)),
            pl.BlockSpec((1, 1, F), lambda b, be: (be[b], 0, 0)),
            pl.BlockSpec((1, 1, D), lambda b, be: (be[b], 0, 0)),
            pl.BlockSpec((BLK, 1), lambda b, be: (b, 0)),
        ],
        out_specs=pl.BlockSpec((BLK, D), lambda b, be: (b, 0)),
    )
    out_sorted = pl.pallas_call(
        _ffn_body,
        grid_spec=grid_spec,
        out_shape=jax.ShapeDtypeStruct((PAD_T, D), jnp.float32),
        compiler_params=pltpu.CompilerParams(
            dimension_semantics=("parallel",),
        ),
    )(bexp, x_sorted, w1, w2, w3, b1r, b2r, b3r, ws2)

    scatter = pl.kernel(
        _scatter_body,
        out_type=jax.ShapeDtypeStruct((T + 8, D), jnp.float32),
        mesh=_vector_mesh,
        scratch_types=[
            pltpu.VMEM((2, HR), jnp.int32),
            pltpu.VMEM((2, HR, D), jnp.float32),
            pltpu.SemaphoreType.DMA,
            pltpu.SemaphoreType.DMA,
        ],
        compiler_params=_sc_params,
    )
    out_padded = scatter(out_sorted, dst_ids)

    final = out_padded[:T].reshape(1, T, D)
    aux_loss = jnp.asarray(0.0, dtype=jnp.float32)
    return (final, aux_loss)


def kernel(x, gate_w, gate_b, w1, b1, w2, b2, w3, b3):
    return _moe(x, gate_w, gate_b, w1, b1, w2, b2, w3, b3)


# speedup vs baseline: 1.5407x; 1.1458x over previous
import dataclasses
import functools

import jax
import jax.numpy as jnp
from jax import lax
from jax.experimental import pallas as pl
from jax.experimental.pallas import tpu as pltpu
from jax.experimental.pallas import tpu_sc as plsc

T = 2048
D = 1024
F = 2048
E = 8

BLK = 128
NB = T // BLK + E
PAD_T = NB * BLK
NC, NS, L = 2, 16, 16
NW = NC * NS
CH = PAD_T // NW
HR = CH // 2
NCHUNK = T // L

_vector_mesh = plsc.VectorSubcoreMesh(core_axis_name="c", subcore_axis_name="s")

_sc_params = pltpu.CompilerParams()
if "needs_layout_passes" in pltpu.CompilerParams.__dataclass_fields__:
    _sc_params = dataclasses.replace(_sc_params, needs_layout_passes=False)


def _gating_body(x_ref, gw_ref, gb_ref, top_ref, wt_ref):
    logits = jnp.dot(x_ref[...], gw_ref[...], preferred_element_type=jnp.float32)
    logits = logits + gb_ref[...]
    m = jnp.max(logits, axis=-1, keepdims=True)
    ssum = jnp.sum(jnp.exp(logits - m), axis=-1)
    top = jnp.argmax(logits, axis=-1).astype(jnp.int32)
    top_ref[...] = top[None, :]
    wt_ref[...] = (1.0 / ssum)[None, :]


def _route_gather_body(top_hbm, wt_hbm, x_hbm,
                       xs_hbm, dst_hbm, ws_hbm, be_hbm,
                       top_v, w_v, src_v, dst_v, ws_v, be_v, s_v, rows_v,
                       sem, sem2):
    wid = lax.axis_index("s") * NC + lax.axis_index("c")

    pltpu.sync_copy(top_hbm, top_v)
    pltpu.sync_copy(wt_hbm, w_v)

    zeros = jnp.zeros((L,), jnp.int32)
    trash = jnp.full((L,), T, jnp.int32)
    fzeros = jnp.zeros((L,), jnp.float32)

    @pl.loop(0, PAD_T, step=L)
    def _(i):
        src_v[pl.ds(i, L)] = zeros
        dst_v[pl.ds(i, L)] = trash
        ws_v[pl.ds(i, L)] = fzeros

    def count_body(i, accs):
        v = top_v[pl.ds(i * L, L)]
        return tuple(
            accs[e] + (v == e).astype(jnp.int32) for e in range(E)
        )

    accs = lax.fori_loop(0, NCHUNK, count_body,
                         tuple(jnp.zeros((L,), jnp.int32) for _ in range(E)))
    counts = [jnp.sum(accs[e]) for e in range(E)]

    bases = []
    ends = []
    cum_blocks = jnp.int32(0)
    for e in range(E):
        bases.append(cum_blocks * BLK)
        cum_blocks = cum_blocks + (counts[e] + (BLK - 1)) // BLK
        ends.append(cum_blocks)

    biota = lax.iota(jnp.int32, L)
    for chunk in range(2):
        bvec = biota + chunk * L
        acc = jnp.zeros((L,), jnp.int32)
        for e in range(E):
            acc = acc + (bvec >= ends[e]).astype(jnp.int32)
        bev = jnp.minimum(acc, E - 1)
        if chunk == 1:
            bev = jnp.where(biota == L - 1, cum_blocks, bev)
        be_v[pl.ds(chunk * L, L)] = bev

    base_vec = jnp.zeros((L,), jnp.int32)
    for e in range(E):
        base_vec = jnp.where(biota == e, bases[e], base_vec)
    s_v[...] = base_vec

    @pl.loop(0, NCHUNK)
    def _(i):
        v = top_v[pl.ds(i * L, L)]
        wv = w_v[pl.ds(i * L, L)]
        tok = lax.iota(jnp.int32, L) + i * L
        g = plsc.load_gather(s_v, [v])
        r, last = plsc.scan_count(v)
        pos = g + r - 1
        plsc.store_scatter(src_v, [pos], tok)
        plsc.store_scatter(dst_v, [pos], tok)
        plsc.store_scatter(ws_v, [pos], wv)
        plsc.store_scatter(s_v, [v], pos + 1, mask=last)

    base = wid * CH
    used_pad = cum_blocks * BLK
    c0 = base < used_pad
    c1 = base + HR < used_pad

    g0 = pltpu.make_async_copy(x_hbm.at[src_v.at[pl.ds(base, HR)]],
                               rows_v.at[0], sem)
    g1 = pltpu.make_async_copy(x_hbm.at[src_v.at[pl.ds(base + HR, HR)]],
                               rows_v.at[1], sem2)
    w0 = pltpu.make_async_copy(rows_v.at[0], xs_hbm.at[pl.ds(base, HR)], sem)
    w1 = pltpu.make_async_copy(rows_v.at[1], xs_hbm.at[pl.ds(base + HR, HR)],
                               sem2)

    @pl.when(c0)
    def _():
        g0.start()

    @pl.when(c1)
    def _():
        g1.start()

    @pl.when(c0)
    def _():
        g0.wait()
        w0.start()

    @pl.when(c1)
    def _():
        g1.wait()
        w1.start()

    pltpu.sync_copy(dst_v.at[pl.ds(base, CH)], dst_hbm.at[pl.ds(base, CH)])
    pltpu.sync_copy(ws_v.at[pl.ds(base, CH)], ws_hbm.at[pl.ds(base, CH)])

    @pl.when(c0)
    def _():
        w0.wait()

    @pl.when(c1)
    def _():
        w1.wait()

    @pl.when(wid == 0)
    def _():
        pltpu.sync_copy(be_v, be_hbm)


def _gelu_exact(v):
    return 0.5 * v * (1.0 + lax.erf(v * 0.7071067811865476))


def _ffn_body(be_ref, x_ref, w1_ref, w2_ref, w3_ref, b1_ref, b2_ref, b3_ref,
              ws_ref, out_ref):
    @pl.when(pl.program_id(0) < be_ref[2 * L - 1])
    def _():
        xb = x_ref[...]
        h1 = jnp.dot(xb, w1_ref[0],
                     preferred_element_type=jnp.float32) + b1_ref[0]
        h2 = jnp.dot(xb, w2_ref[0],
                     preferred_element_type=jnp.float32) + b2_ref[0]
        h = _gelu_exact(h1) * h2
        o = jnp.dot(h, w3_ref[0], preferred_element_type=jnp.float32) + b3_ref[0]
        out_ref[...] = o * ws_ref[...]


def _scatter_body(osort_hbm, dst_hbm, be_hbm, out_hbm, idx_v, rows_v, be_v,
                  sem, sem2):
    wid = lax.axis_index("s") * NC + lax.axis_index("c")
    base = wid * CH
    pltpu.sync_copy(be_hbm, be_v)
    biota = lax.iota(jnp.int32, L)
    hi = be_v[pl.ds(L, L)]
    used_pad = jnp.sum(jnp.where(biota == L - 1, hi, 0)) * BLK
    c0 = base < used_pad
    c1 = base + HR < used_pad

    r0 = pltpu.make_async_copy(osort_hbm.at[pl.ds(base, HR)], rows_v.at[0],
                               sem)
    r1 = pltpu.make_async_copy(osort_hbm.at[pl.ds(base + HR, HR)],
                               rows_v.at[1], sem2)
    s0 = pltpu.make_async_copy(rows_v.at[0], out_hbm.at[idx_v.at[0]], sem)
    s1 = pltpu.make_async_copy(rows_v.at[1], out_hbm.at[idx_v.at[1]], sem2)

    @pl.when(c0)
    def _():
        r0.start()

    @pl.when(c1)
    def _():
        r1.start()

    @pl.when(c0)
    def _():
        pltpu.sync_copy(dst_hbm.at[pl.ds(base, HR)], idx_v.at[0])
        r0.wait()
        s0.start()

    @pl.when(c1)
    def _():
        pltpu.sync_copy(dst_hbm.at[pl.ds(base + HR, HR)], idx_v.at[1])
        r1.wait()
        s1.start()

    @pl.when(c0)
    def _():
        s0.wait()

    @pl.when(c1)
    def _():
        s1.wait()


@jax.jit
def _moe(x, gate_w, gate_b, w1, b1, w2, b2, w3, b3):
    xt = x.reshape(T, D)
    gb = gate_b.reshape(1, E)

    top, wt = pl.pallas_call(
        _gating_body,
        grid=(8,),
        out_shape=(
            jax.ShapeDtypeStruct((1, T), jnp.int32),
            jax.ShapeDtypeStruct((1, T), jnp.float32),
        ),
        in_specs=[
            pl.BlockSpec((T // 8, D), lambda t: (t, 0)),
            pl.BlockSpec((D, E), lambda t: (0, 0)),
            pl.BlockSpec((1, E), lambda t: (0, 0)),
        ],
        out_specs=(
            pl.BlockSpec((1, T // 8), lambda t: (0, t)),
            pl.BlockSpec((1, T // 8), lambda t: (0, t)),
        ),
        compiler_params=pltpu.CompilerParams(
            dimension_semantics=("parallel",),
        ),
    )(xt, gate_w, gb)

    route = pl.kernel(
        _route_gather_body,
        out_type=(
            jax.ShapeDtypeStruct((PAD_T, D), jnp.float32),
            jax.ShapeDtypeStruct((PAD_T,), jnp.int32),
            jax.ShapeDtypeStruct((PAD_T,), jnp.float32),
            jax.ShapeDtypeStruct((2 * L,), jnp.int32),
        ),
        mesh=_vector_mesh,
        scratch_types=[
            pltpu.VMEM((T,), jnp.int32),
            pltpu.VMEM((T,), jnp.float32),
            pltpu.VMEM((PAD_T,), jnp.int32),
            pltpu.VMEM((PAD_T,), jnp.int32),
            pltpu.VMEM((PAD_T,), jnp.float32),
            pltpu.VMEM((2 * L,), jnp.int32),
            pltpu.VMEM((L,), jnp.int32),
            pltpu.VMEM((2, HR, D), jnp.float32),
            pltpu.SemaphoreType.DMA,
            pltpu.SemaphoreType.DMA,
        ],
        compiler_params=_sc_params,
    )
    x_sorted, dst_ids, ws, bexp = route(top.reshape(T), wt.reshape(T), xt)

    b1r = b1.reshape(E, 1, F)
    b2r = b2.reshape(E, 1, F)
    b3r = b3.reshape(E, 1, D)
    ws2 = ws.reshape(PAD_T, 1)

    grid_spec = pltpu.PrefetchScalarGridSpec(
        num_scalar_prefetch=1,
        grid=(NB,),
        in_specs=[
            pl.BlockSpec((BLK, D), lambda b, be: (b, 0)),
            pl.BlockSpec((1, D, F), lambda b, be: (be[b], 0, 0)),
            pl.BlockSpec((1, D, F), lambda b, be: (be[b], 0, 0)),
            pl.BlockSpec((1, F, D), lambda b, be: (be[b], 0, 0)),
            pl.BlockSpec((1, 1, F), lambda b, be: (be[b], 0, 0)),
            pl.BlockSpec((1, 1, F), lambda b, be: (be[b], 0, 0)),
            pl.BlockSpec((1, 1, D), lambda b, be: (be[b], 0, 0)),
            pl.BlockSpec((BLK, 1), lambda b, be: (b, 0)),
        ],
        out_specs=pl.BlockSpec((BLK, D), lambda b, be: (b, 0)),
    )
    out_sorted = pl.pallas_call(
        _ffn_body,
        grid_spec=grid_spec,
        out_shape=jax.ShapeDtypeStruct((PAD_T, D), jnp.float32),
        compiler_params=pltpu.CompilerParams(
            dimension_semantics=("parallel",),
        ),
    )(bexp, x_sorted, w1, w2, w3, b1r, b2r, b3r, ws2)

    scatter = pl.kernel(
        _scatter_body,
        out_type=jax.ShapeDtypeStruct((T + 8, D), jnp.float32),
        mesh=_vector_mesh,
        scratch_types=[
            pltpu.VMEM((2, HR), jnp.int32),
            pltpu.VMEM((2, HR, D), jnp.float32),
            pltpu.VMEM((2 * L,), jnp.int32),
            pltpu.SemaphoreType.DMA,
            pltpu.SemaphoreType.DMA,
        ],
        compiler_params=_sc_params,
    )
    out_padded = scatter(out_sorted, dst_ids, bexp)

    final = out_padded[:T].reshape(1, T, D)
    aux_loss = jnp.asarray(0.0, dtype=jnp.float32)
    return (final, aux_loss)


def kernel(x, gate_w, gate_b, w1, b1, w2, b2, w3, b3):
    return _moe(x, gate_w, gate_b, w1, b1, w2, b2, w3, b3)
